# baseline (speedup 1.0000x reference)
import functools

import jax
import jax.numpy as jnp
from jax import lax
from jax.experimental import pallas as pl
from jax.experimental.pallas import tpu as pltpu
from jax.experimental.pallas import tpu_sc as plsc

_VOCAB = 100000
_EMB = 32
_HID = 64
_BATCH = 1024

_NC = 2
_NS = 16
_NW = _NC * _NS
_B_PER_W = _BATCH // _NW

_TILE_V = 2048


def _sc_gather(emb_table, idx):
    mesh = plsc.VectorSubcoreMesh(
        core_axis_name="c", subcore_axis_name="s", num_cores=_NC, num_subcores=_NS
    )

    @functools.partial(
        pl.kernel,
        out_type=jax.ShapeDtypeStruct((_BATCH, _EMB), jnp.float32),
        mesh=mesh,
        scratch_types=[
            pltpu.VMEM((_B_PER_W,), jnp.int32),
            pltpu.VMEM((_B_PER_W, _EMB), jnp.float32),
            pltpu.SemaphoreType.DMA,
        ],
        compiler_params=pltpu.CompilerParams(use_tc_tiling_on_sc=False),
    )
    def gather_kernel(table_hbm, idx_hbm, out_hbm, idx_v, rows_v, sem):
        wid = lax.axis_index("s") * _NC + lax.axis_index("c")
        base = wid * _B_PER_W
        pltpu.sync_copy(idx_hbm.at[pl.ds(base, _B_PER_W)], idx_v)
        pltpu.async_copy(table_hbm.at[idx_v], rows_v, sem).wait()
        pltpu.sync_copy(rows_v, out_hbm.at[pl.ds(base, _B_PER_W)])

    return gather_kernel(emb_table, idx)


def _mlp_body(emb_ref, w1_ref, b1_ref, w2_ref, b2_ref, out_ref, h_ref):
    @pl.when(pl.program_id(0) == 0)
    def _():
        h = jnp.dot(emb_ref[...], w1_ref[...], preferred_element_type=jnp.float32)
        h_ref[...] = jnp.maximum(h + b1_ref[...], 0.0)

    out_ref[...] = (
        jnp.dot(h_ref[...], w2_ref[...], preferred_element_type=jnp.float32)
        + b2_ref[...]
    )


def _tc_mlp(emb, w1, b1, w2, b2):
    num_tiles = pl.cdiv(_VOCAB, _TILE_V)
    return pl.pallas_call(
        _mlp_body,
        grid=(num_tiles,),
        in_specs=[
            pl.BlockSpec((_BATCH, _EMB), lambda i: (0, 0)),
            pl.BlockSpec((_EMB, _HID), lambda i: (0, 0)),
            pl.BlockSpec((1, _HID), lambda i: (0, 0)),
            pl.BlockSpec((_HID, _TILE_V), lambda i: (0, i)),
            pl.BlockSpec((1, _TILE_V), lambda i: (0, i)),
        ],
        out_specs=pl.BlockSpec((_BATCH, _TILE_V), lambda i: (0, i)),
        out_shape=jax.ShapeDtypeStruct((_BATCH, _VOCAB), jnp.float32),
        scratch_shapes=[pltpu.VMEM((_BATCH, _HID), jnp.float32)],
        compiler_params=pltpu.CompilerParams(
            dimension_semantics=("arbitrary",),
        ),
    )(emb, w1, b1.reshape(1, _HID), w2, b2.reshape(1, _VOCAB))


def kernel(x, emb_table, W1, b1, W2, b2):
    emb = _sc_gather(emb_table, x.astype(jnp.int32))
    return _tc_mlp(emb, W1, b1, W2, b2)


# speedup vs baseline: 2.1330x; 2.1330x over previous
import functools

import jax
import jax.numpy as jnp
from jax import lax
from jax.experimental import pallas as pl
from jax.experimental.pallas import tpu as pltpu
from jax.experimental.pallas import tpu_sc as plsc

_VOCAB = 100000
_EMB = 32
_HID = 64
_BATCH = 1024

_NC = 2
_NS = 16
_NW = _NC * _NS
_B_PER_W = _BATCH // _NW

_TILE_V = 4096


def _sc_gather(emb_table, idx):
    mesh = plsc.VectorSubcoreMesh(
        core_axis_name="c", subcore_axis_name="s", num_cores=_NC, num_subcores=_NS
    )

    @functools.partial(
        pl.kernel,
        out_type=jax.ShapeDtypeStruct((_BATCH, _EMB), jnp.float32),
        mesh=mesh,
        scratch_types=[
            pltpu.VMEM((_B_PER_W,), jnp.int32),
            pltpu.VMEM((_B_PER_W, _EMB), jnp.float32),
            pltpu.SemaphoreType.DMA,
        ],
        compiler_params=pltpu.CompilerParams(use_tc_tiling_on_sc=False),
    )
    def gather_kernel(table_hbm, idx_hbm, out_hbm, idx_v, rows_v, sem):
        wid = lax.axis_index("s") * _NC + lax.axis_index("c")
        base = wid * _B_PER_W
        pltpu.sync_copy(idx_hbm.at[pl.ds(base, _B_PER_W)], idx_v)
        pltpu.async_copy(table_hbm.at[idx_v], rows_v, sem).wait()
        pltpu.sync_copy(rows_v, out_hbm.at[pl.ds(base, _B_PER_W)])

    return gather_kernel(emb_table, idx)


def _mlp_body(emb_ref, w1_ref, b1_ref, w2_ref, b2_ref, out_ref, ht_ref):
    @pl.when(pl.program_id(0) == 0)
    def _():
        ht = lax.dot_general(
            w1_ref[...], emb_ref[...],
            (((0,), (1,)), ((), ())),
            preferred_element_type=jnp.float32,
        )
        ht_ref[...] = jnp.maximum(ht + b1_ref[...], 0.0)

    out_ref[...] = (
        lax.dot_general(
            w2_ref[...], ht_ref[...],
            (((0,), (0,)), ((), ())),
            preferred_element_type=jnp.float32,
        )
        + b2_ref[...]
    )


def _tc_mlp(emb, w1, b1, w2, b2):
    num_tiles = pl.cdiv(_VOCAB, _TILE_V)
    out_t = pl.pallas_call(
        _mlp_body,
        grid=(num_tiles,),
        in_specs=[
            pl.BlockSpec((_BATCH, _EMB), lambda i: (0, 0)),
            pl.BlockSpec((_EMB, _HID), lambda i: (0, 0)),
            pl.BlockSpec((_HID, 1), lambda i: (0, 0)),
            pl.BlockSpec((_HID, _TILE_V), lambda i: (0, i)),
            pl.BlockSpec((_TILE_V, 1), lambda i: (i, 0)),
        ],
        out_specs=pl.BlockSpec((_TILE_V, _BATCH), lambda i: (i, 0)),
        out_shape=jax.ShapeDtypeStruct((_VOCAB, _BATCH), jnp.float32),
        scratch_shapes=[pltpu.VMEM((_HID, _BATCH), jnp.float32)],
        compiler_params=pltpu.CompilerParams(
            dimension_semantics=("arbitrary",),
        ),
    )(emb, w1, b1.reshape(_HID, 1), w2, b2.reshape(_VOCAB, 1))
    return out_t.T


def kernel(x, emb_table, W1, b1, W2, b2):
    emb = _sc_gather(emb_table, x.astype(jnp.int32))
    return _tc_mlp(emb, W1, b1, W2, b2)


# speedup vs baseline: 2.7286x; 1.2792x over previous
import functools

import jax
import jax.numpy as jnp
from jax import lax
from jax.experimental import pallas as pl
from jax.experimental.pallas import tpu as pltpu
from jax.experimental.pallas import tpu_sc as plsc

_VOCAB = 100000
_EMB = 32
_HID = 64
_BATCH = 1024

_NC = 2
_NS = 16
_NW = _NC * _NS
_B_PER_W = _BATCH // _NW

_TILE_V = 4096


def _sc_gather(emb_table, idx):
    mesh = plsc.VectorSubcoreMesh(
        core_axis_name="c", subcore_axis_name="s", num_cores=_NC, num_subcores=_NS
    )

    @functools.partial(
        pl.kernel,
        out_type=jax.ShapeDtypeStruct((_BATCH, _EMB), jnp.float32),
        mesh=mesh,
        scratch_types=[
            pltpu.VMEM((_B_PER_W,), jnp.int32),
            pltpu.VMEM((_B_PER_W, _EMB), jnp.float32),
            pltpu.SemaphoreType.DMA,
        ],
        compiler_params=pltpu.CompilerParams(use_tc_tiling_on_sc=False),
    )
    def gather_kernel(table_hbm, idx_hbm, out_hbm, idx_v, rows_v, sem):
        wid = lax.axis_index("s") * _NC + lax.axis_index("c")
        base = wid * _B_PER_W
        pltpu.sync_copy(idx_hbm.at[pl.ds(base, _B_PER_W)], idx_v)
        pltpu.async_copy(table_hbm.at[idx_v], rows_v, sem).wait()
        pltpu.sync_copy(rows_v, out_hbm.at[pl.ds(base, _B_PER_W)])

    return gather_kernel(emb_table, idx)


def _mlp_body(emb_ref, w1_ref, b1_ref, w2_ref, b2_ref, out_ref, ht_ref):
    @pl.when(pl.program_id(0) == 0)
    def _():
        ht = lax.dot_general(
            w1_ref[...], emb_ref[...],
            (((0,), (1,)), ((), ())),
            preferred_element_type=jnp.float32,
        )
        ht_ref[...] = jnp.maximum(ht + b1_ref[...], 0.0)

    out_ref[...] = (
        lax.dot_general(
            w2_ref[...], ht_ref[...],
            (((0,), (0,)), ((), ())),
            preferred_element_type=jnp.float32,
        )
        + b2_ref[...].T
    )


def _tc_mlp(emb, w1, b1, w2, b2):
    num_tiles = pl.cdiv(_VOCAB, _TILE_V)
    out_t = pl.pallas_call(
        _mlp_body,
        grid=(num_tiles,),
        in_specs=[
            pl.BlockSpec((_BATCH, _EMB), lambda i: (0, 0)),
            pl.BlockSpec((_EMB, _HID), lambda i: (0, 0)),
            pl.BlockSpec((_HID, 1), lambda i: (0, 0)),
            pl.BlockSpec((_HID, _TILE_V), lambda i: (0, i)),
            pl.BlockSpec((1, _TILE_V), lambda i: (0, i)),
        ],
        out_specs=pl.BlockSpec((_TILE_V, _BATCH), lambda i: (i, 0)),
        out_shape=jax.ShapeDtypeStruct((_VOCAB, _BATCH), jnp.float32),
        scratch_shapes=[pltpu.VMEM((_HID, _BATCH), jnp.float32)],
        compiler_params=pltpu.CompilerParams(
            dimension_semantics=("arbitrary",),
        ),
    )(emb, w1, b1.reshape(_HID, 1), w2, b2.reshape(1, _VOCAB))
    return out_t.T


def kernel(x, emb_table, W1, b1, W2, b2):
    emb = _sc_gather(emb_table, x.astype(jnp.int32))
    return _tc_mlp(emb, W1, b1, W2, b2)


# speedup vs baseline: 3.0634x; 1.1227x over previous
import functools

import jax
import jax.numpy as jnp
from jax import lax
from jax.experimental import pallas as pl
from jax.experimental.pallas import tpu as pltpu
from jax.experimental.pallas import tpu_sc as plsc

_VOCAB = 100000
_EMB = 32
_HID = 64
_BATCH = 1024

_NC = 2
_NS = 16
_NW = _NC * _NS
_B_PER_W = _BATCH // _NW

_TILE_V = 4096


def _sc_gather(emb_table, idx):
    mesh = plsc.VectorSubcoreMesh(
        core_axis_name="c", subcore_axis_name="s", num_cores=_NC, num_subcores=_NS
    )

    @functools.partial(
        pl.kernel,
        out_type=jax.ShapeDtypeStruct((_BATCH, _EMB), jnp.float32),
        mesh=mesh,
        scratch_types=[
            pltpu.VMEM((_B_PER_W,), jnp.int32),
            pltpu.VMEM((_B_PER_W, _EMB), jnp.float32),
            pltpu.SemaphoreType.DMA,
        ],
        compiler_params=pltpu.CompilerParams(
            use_tc_tiling_on_sc=True, needs_layout_passes=False
        ),
    )
    def gather_kernel(table_hbm, idx_hbm, out_hbm, idx_v, rows_v, sem):
        wid = lax.axis_index("s") * _NC + lax.axis_index("c")
        base = wid * _B_PER_W
        pltpu.sync_copy(idx_hbm.at[pl.ds(base, _B_PER_W)], idx_v)
        lane = lax.iota(jnp.int32, 16)
        copies = []
        for h in range(_B_PER_W // 16):
            chunk = idx_v[pl.ds(h * 16, 16)]
            for j in range(16):
                r = jnp.max(jnp.where(lane == j, chunk, -1))
                copies.append(
                    pltpu.make_async_copy(
                        table_hbm.at[pl.ds(r, 1)],
                        rows_v.at[pl.ds(h * 16 + j, 1)],
                        sem,
                    )
                )
        for c in copies:
            c.start()
        for c in copies:
            c.wait()
        pltpu.sync_copy(rows_v, out_hbm.at[pl.ds(base, _B_PER_W)])

    return gather_kernel(emb_table, idx)


def _mlp_body(emb_ref, w1_ref, b1_ref, w2_ref, b2_ref, out_ref, ht_ref):
    @pl.when(pl.program_id(0) == 0)
    def _():
        ht = lax.dot_general(
            w1_ref[...], emb_ref[...],
            (((0,), (1,)), ((), ())),
            preferred_element_type=jnp.float32,
        )
        ht_ref[...] = jnp.maximum(ht + b1_ref[...], 0.0)

    out_ref[...] = (
        lax.dot_general(
            w2_ref[...], ht_ref[...],
            (((0,), (0,)), ((), ())),
            preferred_element_type=jnp.float32,
        )
        + b2_ref[...].T
    )


def _tc_mlp(emb, w1, b1, w2, b2):
    num_tiles = pl.cdiv(_VOCAB, _TILE_V)
    out_t = pl.pallas_call(
        _mlp_body,
        grid=(num_tiles,),
        in_specs=[
            pl.BlockSpec((_BATCH, _EMB), lambda i: (0, 0)),
            pl.BlockSpec((_EMB, _HID), lambda i: (0, 0)),
            pl.BlockSpec((_HID, 1), lambda i: (0, 0)),
            pl.BlockSpec((_HID, _TILE_V), lambda i: (0, i)),
            pl.BlockSpec((1, _TILE_V), lambda i: (0, i)),
        ],
        out_specs=pl.BlockSpec((_TILE_V, _BATCH), lambda i: (i, 0)),
        out_shape=jax.ShapeDtypeStruct((_VOCAB, _BATCH), jnp.float32),
        scratch_shapes=[pltpu.VMEM((_HID, _BATCH), jnp.float32)],
        compiler_params=pltpu.CompilerParams(
            dimension_semantics=("arbitrary",),
        ),
    )(emb, w1, b1.reshape(_HID, 1), w2, b2.reshape(1, _VOCAB))
    return out_t.T


def kernel(x, emb_table, W1, b1, W2, b2):
    emb = _sc_gather(emb_table, x.astype(jnp.int32))
    return _tc_mlp(emb, W1, b1, W2, b2)


# speedup vs baseline: 3.4345x; 1.1211x over previous
import functools

import jax
import jax.numpy as jnp
from jax import lax
from jax.experimental import pallas as pl
from jax.experimental.pallas import tpu as pltpu
from jax.experimental.pallas import tpu_sc as plsc

_VOCAB = 100000
_EMB = 32
_HID = 64
_BATCH = 1024

_NC = 2
_NS = 16
_NW = _NC * _NS
_B_PER_W = _BATCH // _NW

_TILE_V = 4096


_HALF = _B_PER_W // 2


def _sc_gather_cols(table_t, idx):
    mesh = plsc.VectorSubcoreMesh(
        core_axis_name="c", subcore_axis_name="s", num_cores=_NC, num_subcores=_NS
    )

    @functools.partial(
        pl.kernel,
        out_type=jax.ShapeDtypeStruct((_BATCH, _EMB), jnp.float32),
        mesh=mesh,
        scratch_types=[
            pltpu.VMEM((_B_PER_W,), jnp.int32),
            pltpu.VMEM((_HALF, _EMB, 128), jnp.float32),
            pltpu.VMEM((_B_PER_W, _EMB), jnp.float32),
            pltpu.SemaphoreType.DMA,
        ],
        compiler_params=pltpu.CompilerParams(
            use_tc_tiling_on_sc=True, needs_layout_passes=False
        ),
    )
    def gk(table_hbm, idx_hbm, out_hbm, idx_v, blk_v, rows_v, sem):
        wid = lax.axis_index("s") * _NC + lax.axis_index("c")
        base = wid * _B_PER_W
        pltpu.sync_copy(idx_hbm.at[pl.ds(base, _B_PER_W)], idx_v)
        lane = lax.iota(jnp.int32, 16)
        for half in range(2):
            chunk = idx_v[pl.ds(half * 16, 16)]
            copies = []
            for j in range(16):
                r = jnp.max(jnp.where(lane == j, chunk, -1))
                jcol = lax.shift_right_logical(r, 7) * 128
                copies.append(
                    pltpu.make_async_copy(
                        table_hbm.at[:, pl.ds(jcol, 128)],
                        blk_v.at[j],
                        sem,
                    )
                )
            for c in copies:
                c.start()
            for c in copies:
                c.wait()
            lmod = chunk & 127
            for j in range(16):
                lr = jnp.max(jnp.where(lane == j, lmod, -1))
                for h in range(_EMB // 16):
                    svec = h * 16 + lane
                    vals = plsc.load_gather(
                        blk_v, [jnp.zeros((16,), jnp.int32) + j, svec,
                                jnp.zeros((16,), jnp.int32) + lr]
                    )
                    rows_v[half * 16 + j, pl.ds(h * 16, 16)] = vals
        pltpu.sync_copy(rows_v, out_hbm.at[pl.ds(base, _B_PER_W)])

    return gk(table_t, idx)


def _mlp_body(emb_ref, w1_ref, b1_ref, w2_ref, b2_ref, out_ref, ht_ref):
    @pl.when(pl.program_id(0) == 0)
    def _():
        ht = lax.dot_general(
            w1_ref[...], emb_ref[...],
            (((0,), (1,)), ((), ())),
            preferred_element_type=jnp.float32,
        )
        ht_ref[...] = jnp.maximum(ht + b1_ref[...], 0.0)

    out_ref[...] = (
        lax.dot_general(
            w2_ref[...], ht_ref[...],
            (((0,), (0,)), ((), ())),
            preferred_element_type=jnp.float32,
        )
        + b2_ref[...].T
    )


def _tc_mlp(emb, w1, b1, w2, b2):
    num_tiles = pl.cdiv(_VOCAB, _TILE_V)
    out_t = pl.pallas_call(
        _mlp_body,
        grid=(num_tiles,),
        in_specs=[
            pl.BlockSpec((_BATCH, _EMB), lambda i: (0, 0)),
            pl.BlockSpec((_EMB, _HID), lambda i: (0, 0)),
            pl.BlockSpec((_HID, 1), lambda i: (0, 0)),
            pl.BlockSpec((_HID, _TILE_V), lambda i: (0, i)),
            pl.BlockSpec((1, _TILE_V), lambda i: (0, i)),
        ],
        out_specs=pl.BlockSpec((_TILE_V, _BATCH), lambda i: (i, 0)),
        out_shape=jax.ShapeDtypeStruct((_VOCAB, _BATCH), jnp.float32),
        scratch_shapes=[pltpu.VMEM((_HID, _BATCH), jnp.float32)],
        compiler_params=pltpu.CompilerParams(
            dimension_semantics=("arbitrary",),
        ),
    )(emb, w1, b1.reshape(_HID, 1), w2, b2.reshape(1, _VOCAB))
    return out_t.T


def kernel(x, emb_table, W1, b1, W2, b2):
    emb = _sc_gather_cols(emb_table.T, x.astype(jnp.int32))
    return _tc_mlp(emb, W1, b1, W2, b2)


# speedup vs baseline: 3.4372x; 1.0008x over previous
import functools

import jax
import jax.numpy as jnp
from jax import lax
from jax.experimental import pallas as pl
from jax.experimental.pallas import tpu as pltpu
from jax.experimental.pallas import tpu_sc as plsc

_VOCAB = 100000
_EMB = 32
_HID = 64
_BATCH = 1024

_NC = 2
_NS = 16
_NW = _NC * _NS
_B_PER_W = _BATCH // _NW

_TILE_V = 4096


_HALF = _B_PER_W // 2


def _sc_gather_cols(table_t, idx):
    mesh = plsc.VectorSubcoreMesh(
        core_axis_name="c", subcore_axis_name="s", num_cores=_NC, num_subcores=_NS
    )

    @functools.partial(
        pl.kernel,
        out_type=jax.ShapeDtypeStruct((_BATCH, _EMB), jnp.float32),
        mesh=mesh,
        scratch_types=[
            pltpu.VMEM((_B_PER_W,), jnp.int32),
            pltpu.VMEM((8, _EMB, 128), jnp.float32),
            pltpu.VMEM((8, _EMB, 128), jnp.float32),
            pltpu.VMEM((_B_PER_W, _EMB), jnp.float32),
            pltpu.SemaphoreType.DMA,
            pltpu.SemaphoreType.DMA,
        ],
        compiler_params=pltpu.CompilerParams(
            use_tc_tiling_on_sc=True, needs_layout_passes=False
        ),
    )
    def gk(table_hbm, idx_hbm, out_hbm, idx_v, blk_a, blk_b, rows_v, sem_a, sem_b):
        wid = lax.axis_index("s") * _NC + lax.axis_index("c")
        base = wid * _B_PER_W
        pltpu.sync_copy(idx_hbm.at[pl.ds(base, _B_PER_W)], idx_v)
        lane = lax.iota(jnp.int32, 16)
        bufs = (blk_a, blk_b)
        sems = (sem_a, sem_b)

        def fire(q):
            chunk = idx_v[pl.ds((q // 2) * 16, 16)]
            copies = []
            for j in range(8):
                jj = (q % 2) * 8 + j
                r = jnp.max(jnp.where(lane == jj, chunk, -1))
                jcol = lax.shift_right_logical(r, 7) * 128
                copies.append(
                    pltpu.make_async_copy(
                        table_hbm.at[:, pl.ds(jcol, 128)],
                        bufs[q % 2].at[j],
                        sems[q % 2],
                    )
                )
            for c in copies:
                c.start()
            return copies

        def extract(q):
            lmod = idx_v[pl.ds((q // 2) * 16, 16)] & 127
            for j in range(8):
                jj = (q % 2) * 8 + j
                lr = jnp.max(jnp.where(lane == jj, lmod, -1))
                for h in range(_EMB // 16):
                    svec = h * 16 + lane
                    vals = plsc.load_gather(
                        bufs[q % 2], [jnp.zeros((16,), jnp.int32) + j, svec,
                                      jnp.zeros((16,), jnp.int32) + lr]
                    )
                    rows_v[q * 8 + j, pl.ds(h * 16, 16)] = vals

        inflight = [fire(0), fire(1)]
        for q in range(4):
            for c in inflight[q]:
                c.wait()
            if q + 2 < 4:
                inflight.append(None)
            extract(q)
            if q + 2 < 4:
                inflight[q + 2] = fire(q + 2)
        pltpu.sync_copy(rows_v, out_hbm.at[pl.ds(base, _B_PER_W)])

    return gk(table_t, idx)


def _mlp_body(emb_ref, w1_ref, b1_ref, w2_ref, b2_ref, out_ref, ht_ref):
    @pl.when(pl.program_id(0) == 0)
    def _():
        ht = lax.dot_general(
            w1_ref[...], emb_ref[...],
            (((0,), (1,)), ((), ())),
            preferred_element_type=jnp.float32,
        )
        ht_ref[...] = jnp.maximum(ht + b1_ref[...], 0.0)

    out_ref[...] = (
        lax.dot_general(
            w2_ref[...], ht_ref[...],
            (((0,), (0,)), ((), ())),
            preferred_element_type=jnp.float32,
        )
        + b2_ref[...].T
    )


def _tc_mlp(emb, w1, b1, w2, b2):
    num_tiles = pl.cdiv(_VOCAB, _TILE_V)
    out_t = pl.pallas_call(
        _mlp_body,
        grid=(num_tiles,),
        in_specs=[
            pl.BlockSpec((_BATCH, _EMB), lambda i: (0, 0)),
            pl.BlockSpec((_EMB, _HID), lambda i: (0, 0)),
            pl.BlockSpec((_HID, 1), lambda i: (0, 0)),
            pl.BlockSpec((_HID, _TILE_V), lambda i: (0, i)),
            pl.BlockSpec((1, _TILE_V), lambda i: (0, i)),
        ],
        out_specs=pl.BlockSpec((_TILE_V, _BATCH), lambda i: (i, 0)),
        out_shape=jax.ShapeDtypeStruct((_VOCAB, _BATCH), jnp.float32),
        scratch_shapes=[pltpu.VMEM((_HID, _BATCH), jnp.float32)],
        compiler_params=pltpu.CompilerParams(
            dimension_semantics=("arbitrary",),
        ),
    )(emb, w1, b1.reshape(_HID, 1), w2, b2.reshape(1, _VOCAB))
    return out_t.T


def kernel(x, emb_table, W1, b1, W2, b2):
    emb = _sc_gather_cols(emb_table.T, x.astype(jnp.int32))
    return _tc_mlp(emb, W1, b1, W2, b2)


# speedup vs baseline: 3.4843x; 1.0137x over previous
import functools

import jax
import jax.numpy as jnp
from jax import lax
from jax.experimental import pallas as pl
from jax.experimental.pallas import tpu as pltpu
from jax.experimental.pallas import tpu_sc as plsc

_VOCAB = 100000
_EMB = 32
_HID = 64
_BATCH = 1024

_NC = 2
_NS = 16
_NW = _NC * _NS
_B_PER_W = _BATCH // _NW

_TILE_V = 4096


_HALF = _B_PER_W // 2


def _sc_gather_cols(table_t, idx):
    mesh = plsc.VectorSubcoreMesh(
        core_axis_name="c", subcore_axis_name="s", num_cores=_NC, num_subcores=_NS
    )

    @functools.partial(
        pl.kernel,
        out_type=jax.ShapeDtypeStruct((_BATCH, _EMB), jnp.float32),
        mesh=mesh,
        scratch_types=[
            pltpu.VMEM((_B_PER_W,), jnp.int32),
            pltpu.VMEM((8, _EMB, 128), jnp.float32),
            pltpu.VMEM((8, _EMB, 128), jnp.float32),
            pltpu.VMEM((_B_PER_W, _EMB), jnp.float32),
            pltpu.SemaphoreType.DMA,
            pltpu.SemaphoreType.DMA,
        ],
        compiler_params=pltpu.CompilerParams(
            use_tc_tiling_on_sc=True, needs_layout_passes=False
        ),
    )
    def gk(table_hbm, idx_hbm, out_hbm, idx_v, blk_a, blk_b, rows_v, sem_a, sem_b):
        wid = lax.axis_index("s") * _NC + lax.axis_index("c")
        base = wid * _B_PER_W
        pltpu.sync_copy(idx_hbm.at[pl.ds(base, _B_PER_W)], idx_v)
        lane = lax.iota(jnp.int32, 16)
        bufs = (blk_a, blk_b)
        sems = (sem_a, sem_b)

        def fire(q):
            chunk = idx_v[pl.ds((q // 2) * 16, 16)]
            copies = []
            for j in range(8):
                jj = (q % 2) * 8 + j
                r = jnp.max(jnp.where(lane == jj, chunk, -1))
                jcol = lax.shift_right_logical(r, 7) * 128
                copies.append(
                    pltpu.make_async_copy(
                        table_hbm.at[:, pl.ds(jcol, 128)],
                        bufs[q % 2].at[j],
                        sems[q % 2],
                    )
                )
            for c in copies:
                c.start()
            return copies

        def extract(q):
            lmod = idx_v[pl.ds((q // 2) * 16, 16)] & 127
            for j in range(8):
                jj = (q % 2) * 8 + j
                lr = jnp.max(jnp.where(lane == jj, lmod, -1))
                for h in range(_EMB // 16):
                    svec = h * 16 + lane
                    vals = plsc.load_gather(
                        bufs[q % 2], [jnp.zeros((16,), jnp.int32) + j, svec,
                                      jnp.zeros((16,), jnp.int32) + lr]
                    )
                    rows_v[q * 8 + j, pl.ds(h * 16, 16)] = vals

        inflight = {0: fire(0), 1: fire(1)}
        for q in range(4):
            for c in inflight[q]:
                c.wait()
            extract(q)
            if q + 2 < 4:
                inflight[q + 2] = fire(q + 2)
        pltpu.sync_copy(rows_v, out_hbm.at[pl.ds(base, _B_PER_W)])

    return gk(table_t, idx)


def _mlp_body(emb_ref, w1_ref, b1_ref, w2_ref, b2_ref, out_ref, ht_ref):
    @pl.when(pl.program_id(0) == 0)
    def _():
        ht = lax.dot_general(
            w1_ref[...], emb_ref[...],
            (((0,), (1,)), ((), ())),
            preferred_element_type=jnp.float32,
        )
        ht_ref[...] = jnp.maximum(ht + b1_ref[...], 0.0)

    out_ref[...] = (
        lax.dot_general(
            w2_ref[...], ht_ref[...],
            (((0,), (0,)), ((), ())),
            preferred_element_type=jnp.float32,
        )
        + b2_ref[...].T
    )


def _tc_mlp(emb, w1, b1, w2, b2):
    num_tiles = pl.cdiv(_VOCAB, _TILE_V)
    out_t = pl.pallas_call(
        _mlp_body,
        grid=(num_tiles,),
        in_specs=[
            pl.BlockSpec((_BATCH, _EMB), lambda i: (0, 0)),
            pl.BlockSpec((_EMB, _HID), lambda i: (0, 0)),
            pl.BlockSpec((_HID, 1), lambda i: (0, 0)),
            pl.BlockSpec((_HID, _TILE_V), lambda i: (0, i)),
            pl.BlockSpec((1, _TILE_V), lambda i: (0, i)),
        ],
        out_specs=pl.BlockSpec((_TILE_V, _BATCH), lambda i: (i, 0)),
        out_shape=jax.ShapeDtypeStruct((_VOCAB, _BATCH), jnp.float32),
        scratch_shapes=[pltpu.VMEM((_HID, _BATCH), jnp.float32)],
        compiler_params=pltpu.CompilerParams(
            dimension_semantics=("arbitrary",),
        ),
    )(emb, w1, b1.reshape(_HID, 1), w2, b2.reshape(1, _VOCAB))
    return out_t.T


def kernel(x, emb_table, W1, b1, W2, b2):
    emb = _sc_gather_cols(emb_table.T, x.astype(jnp.int32))
    return _tc_mlp(emb, W1, b1, W2, b2)
